# baseline (speedup 1.0000x reference)
import numpy as np
import jax
import jax.numpy as jnp
from jax import lax
from jax.experimental import pallas as pl

B, N, D = 1, 2048, 1024
H, KH = 16, 2
G = H // KH
DH = D // H
CBS = 32
SBS = 32
NSEL = 4
WIN = 256
NMEM = 2
W = N // CBS
SCALE = DH ** -0.5
NEG = -1e9
LN2 = float(np.log(2.0))
S1 = NSEL + 1
RB = 256
NB = N // RB
JPAD = 128
GR = G * RB
KT = 512

_PERM = np.concatenate([np.arange(0, DH, 2), np.arange(1, DH, 2)])
_GPERM = np.array([h * 3 + j for j in range(3) for h in range(H)])


def _proj_kernel(x_ref, nw_ref, w_ref, bs_ref, cos_ref, sin_ref,
                 q_ref, k_ref, kpre_ref, v_ref, vb_ref, g_ref):
    xb = x_ref[...]
    ms = jnp.mean(xb * xb, axis=1, keepdims=True)
    xn = xb * lax.rsqrt(ms + 1e-6) * nw_ref[...]
    y = jnp.dot(xn, w_ref[...], preferred_element_type=jnp.float32)
    c = cos_ref[...]
    s = sin_ref[...]
    for h in range(H):
        a = y[:, h * DH:h * DH + DH // 2]
        b = y[:, h * DH + DH // 2:(h + 1) * DH]
        q_ref[h // G, h % G] = jnp.concatenate(
            [a * c - b * s, b * c + a * s], axis=1)
    for h in range(KH):
        base = H * DH + h * DH
        a = y[:, base:base + DH // 2]
        b = y[:, base + DH // 2:base + DH]
        k_ref[h] = jnp.concatenate([a * c - b * s, b * c + a * s], axis=1)
    kp0 = H * DH + KH * DH
    for h in range(KH):
        kpre_ref[h] = y[:, kp0 + h * DH:kp0 + (h + 1) * DH]
        vpiece = y[:, kp0 + KH * DH + h * DH:kp0 + KH * DH + (h + 1) * DH]
        v_ref[h] = vpiece
        vb_ref[h] = vpiece.astype(jnp.bfloat16)
    g0 = kp0 + 2 * KH * DH
    gy = jax.nn.sigmoid(y[:, g0:] + bs_ref[...])
    for j in range(3):
        for h in range(KH):
            g_ref[j, h] = gy[:, j * H + h * G:j * H + (h + 1) * G]


def _mlp_kernel(x_ref, pos_ref, w1_ref, b1_ref, w2_ref, b2_ref, o_ref):
    pos = jnp.concatenate(
        [jnp.broadcast_to(pos_ref[h:h + 1], (W, CBS * DH)) for h in range(KH)],
        axis=0)
    xp = x_ref[...] + pos
    hdn = jnp.maximum(
        jnp.dot(xp, w1_ref[...], preferred_element_type=jnp.float32)
        + b1_ref[...], 0.0)
    o_ref[...] = jnp.dot(hdn, w2_ref[...],
                         preferred_element_type=jnp.float32) + b2_ref[...]


def _mega_kernel(q_ref, k_ref, v_ref, ck_ref, cv_ref, gt_ref, wo_ref, o_ref):
    qb = pl.program_id(0)
    kh = pl.program_id(1)
    q2 = q_ref[0].reshape(GR, DH) * SCALE
    rows = qb * RB + lax.broadcasted_iota(jnp.int32, (RB, 1), 0)

    ck = ck_ref[0]
    cv = cv_ref[0]
    j = lax.broadcasted_iota(jnp.int32, (RB, JPAD), 1)
    cvalid = (j >= JPAD - NMEM) | ((j < W) & (rows >= (j + 1) * CBS - 1))
    csim = jnp.dot(q2, ck.T, preferred_element_type=jnp.float32)
    csim = jnp.where(cvalid[None], csim.reshape(G, RB, JPAD),
                     NEG).reshape(GR, JPAD)
    cmx = jnp.max(csim, axis=1, keepdims=True)
    cp = jnp.exp(csim - cmx)
    cattn = cp / jnp.sum(cp, axis=1, keepdims=True)
    cout = jnp.dot(cattn, cv,
                   preferred_element_type=jnp.float32).reshape(G, RB, DH)
    imp = jnp.sum(cattn.reshape(G, RB, JPAD)[:, :, :W], axis=0)
    lane_w = lax.broadcasted_iota(jnp.int32, (RB, W), 1)
    work = imp
    sels = []
    for _ in range(NSEL):
        mx = jnp.max(work, axis=1, keepdims=True)
        am = jnp.min(jnp.where(work == mx, lane_w, W), axis=1, keepdims=True)
        sels.append(am)
        work = jnp.where(lane_w == am, -jnp.inf, work)
    sels.append(rows // SBS)

    lane_t = lax.broadcasted_iota(jnp.int32, (1, KT), 1)
    nt = (qb + 2) // 2
    e_end = (qb + 1) * RB

    def flash_update(sim, vv, m, l, acc):
        mn = jnp.maximum(m, jnp.max(sim, axis=1, keepdims=True))
        p = jnp.exp(sim - mn).astype(jnp.bfloat16)
        alpha = jnp.exp(m - mn)
        l = l * alpha + jnp.sum(p.astype(jnp.float32), axis=1, keepdims=True)
        acc = acc * alpha + jnp.dot(p, vv, preferred_element_type=jnp.float32)
        return mn, l, acc

    def fine_bias(kpos, limit):
        wl = kpos // SBS
        mult = jnp.zeros((RB, KT), jnp.int32)
        for s_i in range(S1):
            mult = mult + (sels[s_i] == wl).astype(jnp.int32)
        return jnp.where((mult > 0) & (kpos <= rows) & (kpos < limit),
                         jnp.where(mult == 2, LN2, 0.0), NEG)

    def fine_body(i, carry):
        m, l, acc = carry
        start_raw = e_end - KT * (nt - i)
        start = pl.multiple_of(jnp.maximum(start_raw, 0), RB)
        kk = k_ref[0, pl.ds(start, KT), :]
        vv = v_ref[0, pl.ds(start, KT), :]
        sim = jnp.dot(q2, kk.T, preferred_element_type=jnp.float32)
        kpos = start + lane_t
        bias = fine_bias(kpos, start_raw + KT)
        sim = (sim.reshape(G, RB, KT) + bias[None]).reshape(GR, KT)
        return flash_update(sim, vv, m, l, acc)

    init = (jnp.full((GR, 1), -1e30, jnp.float32),
            jnp.zeros((GR, 1), jnp.float32),
            jnp.zeros((GR, DH), jnp.float32))
    m, l, acc = lax.fori_loop(0, nt - 1, fine_body, init)

    start = pl.multiple_of(jnp.maximum(e_end - KT, 0), RB)
    kk = k_ref[0, pl.ds(start, KT), :]
    vv = v_ref[0, pl.ds(start, KT), :]
    tsim = jnp.dot(q2, kk.T, preferred_element_type=jnp.float32)
    kpos = start + lane_t
    bias = fine_bias(kpos, e_end)
    fsim = (tsim.reshape(G, RB, KT) + bias[None]).reshape(GR, KT)
    m, l, acc = flash_update(fsim, vv, m, l, acc)
    fout = (acc / l).reshape(G, RB, DH)

    dgap = rows - kpos
    wbias = jnp.where((dgap >= 0) & (dgap <= WIN), 0.0, NEG)
    wsim = (tsim.reshape(G, RB, KT) + wbias[None]).reshape(GR, KT)
    wmx = jnp.max(wsim, axis=1, keepdims=True)
    wp = jnp.exp(wsim - wmx).astype(jnp.bfloat16)
    sout = (jnp.dot(wp, vv, preferred_element_type=jnp.float32)
            / jnp.sum(wp.astype(jnp.float32), axis=1,
                      keepdims=True)).reshape(G, RB, DH)

    gt = gt_ref[:, 0]
    combs = []
    for g in range(G):
        combs.append(gt[0][:, g:g + 1] * cout[g]
                     + gt[1][:, g:g + 1] * fout[g]
                     + gt[2][:, g:g + 1] * sout[g])
    comb = jnp.concatenate(combs, axis=1)
    part = jnp.dot(comb, wo_ref[0], preferred_element_type=jnp.float32)

    @pl.when(kh == 0)
    def _():
        o_ref[...] = part

    @pl.when(kh != 0)
    def _():
        o_ref[...] += part


def kernel(hidden_states, norm_w, Wq, Wk, Wv, k_pos, v_pos, Wk1, bk1, Wk2,
           bk2, Wv1, bv1, Wv2, bv2, mem_kv, Ws, bs, Wo):
    P = _PERM
    x = hidden_states.reshape(N, D)
    WqP = Wq.reshape(D, H, DH)[:, :, P].reshape(D, H * DH)
    WkP = Wk.reshape(D, KH, DH)[:, :, P].reshape(D, KH * DH)
    Wcat = jnp.concatenate([WqP, WkP, Wk, Wv, Ws[:, _GPERM]], axis=1)
    CW = H * DH + 3 * KH * DH + 3 * H

    posf = jnp.arange(N, dtype=jnp.float32)
    inv = 1.0 / (10000.0 ** (jnp.arange(0, DH, 2, dtype=jnp.float32) / DH))
    ang = posf[:, None] * inv[None, :]
    cosT = jnp.cos(ang)
    sinT = jnp.sin(ang)

    q4, krot, kpre, vkh, vbf, gates = pl.pallas_call(
        _proj_kernel,
        grid=(NB,),
        in_specs=[
            pl.BlockSpec((RB, D), lambda i: (i, 0)),
            pl.BlockSpec((1, D), lambda i: (0, 0)),
            pl.BlockSpec((D, CW), lambda i: (0, 0)),
            pl.BlockSpec((1, 3 * H), lambda i: (0, 0)),
            pl.BlockSpec((RB, DH // 2), lambda i: (i, 0)),
            pl.BlockSpec((RB, DH // 2), lambda i: (i, 0)),
        ],
        out_specs=[
            pl.BlockSpec((KH, G, RB, DH), lambda i: (0, 0, i, 0)),
            pl.BlockSpec((KH, RB, DH), lambda i: (0, i, 0)),
            pl.BlockSpec((KH, RB, DH), lambda i: (0, i, 0)),
            pl.BlockSpec((KH, RB, DH), lambda i: (0, i, 0)),
            pl.BlockSpec((KH, RB, DH), lambda i: (0, i, 0)),
            pl.BlockSpec((3, KH, RB, G), lambda i: (0, 0, i, 0)),
        ],
        out_shape=[
            jax.ShapeDtypeStruct((KH, G, N, DH), jnp.float32),
            jax.ShapeDtypeStruct((KH, N, DH), jnp.float32),
            jax.ShapeDtypeStruct((KH, N, DH), jnp.float32),
            jax.ShapeDtypeStruct((KH, N, DH), jnp.float32),
            jax.ShapeDtypeStruct((KH, N, DH), jnp.bfloat16),
            jax.ShapeDtypeStruct((3, KH, N, G), jnp.float32),
        ],
    )(x, norm_w.reshape(1, D), Wcat, bs[_GPERM].reshape(1, 3 * H), cosT, sinT)

    def mlp_call(xflat, pos2, W1, b1, W2, b2):
        return pl.pallas_call(
            _mlp_kernel,
            grid=(1,),
            in_specs=[
                pl.BlockSpec((KH * W, CBS * DH), lambda i: (0, 0)),
                pl.BlockSpec((KH, CBS * DH), lambda i: (0, 0)),
                pl.BlockSpec((CBS * DH, CBS * DH), lambda i: (0, 0)),
                pl.BlockSpec((1, CBS * DH), lambda i: (0, 0)),
                pl.BlockSpec((CBS * DH, DH), lambda i: (0, 0)),
                pl.BlockSpec((1, DH), lambda i: (0, 0)),
            ],
            out_specs=pl.BlockSpec((KH * W, DH), lambda i: (0, 0)),
            out_shape=jax.ShapeDtypeStruct((KH * W, DH), jnp.float32),
        )(xflat, pos2, W1, b1.reshape(1, CBS * DH), W2, b2.reshape(1, DH))

    kflat = kpre.reshape(KH * W, CBS * DH)
    vflat = vkh.reshape(KH * W, CBS * DH)
    ck = mlp_call(kflat, k_pos.reshape(KH, CBS * DH), Wk1, bk1, Wk2,
                  bk2).reshape(KH, W, DH)
    cv = mlp_call(vflat, v_pos.reshape(KH, CBS * DH), Wv1, bv1, Wv2,
                  bv2).reshape(KH, W, DH)

    zpad = jnp.zeros((KH, JPAD - NMEM - W, DH), jnp.float32)
    ckf = jnp.concatenate([ck, zpad, mem_kv[0]], axis=1)[:, :, P]
    cvf = jnp.concatenate([cv, zpad, mem_kv[1]], axis=1)

    out = pl.pallas_call(
        _mega_kernel,
        grid=(NB, KH),
        in_specs=[
            pl.BlockSpec((1, G, RB, DH), lambda i, h: (h, 0, i, 0)),
            pl.BlockSpec((1, N, DH), lambda i, h: (h, 0, 0)),
            pl.BlockSpec((1, N, DH), lambda i, h: (h, 0, 0)),
            pl.BlockSpec((1, JPAD, DH), lambda i, h: (h, 0, 0)),
            pl.BlockSpec((1, JPAD, DH), lambda i, h: (h, 0, 0)),
            pl.BlockSpec((3, 1, RB, G), lambda i, h: (0, h, i, 0)),
            pl.BlockSpec((1, G * DH, D), lambda i, h: (h, 0, 0)),
        ],
        out_specs=pl.BlockSpec((RB, D), lambda i, h: (i, 0)),
        out_shape=jax.ShapeDtypeStruct((N, D), jnp.float32),
    )(q4, krot, vbf, ckf, cvf, gates, Wo.reshape(KH, G * DH, D))
    return out.reshape(B, N, D)


# speedup vs baseline: 1.0153x; 1.0153x over previous
import numpy as np
import jax
import jax.numpy as jnp
from jax import lax
from jax.experimental import pallas as pl

B, N, D = 1, 2048, 1024
H, KH = 16, 2
G = H // KH
DH = D // H
CBS = 32
SBS = 32
NSEL = 4
WIN = 256
NMEM = 2
W = N // CBS
SCALE = DH ** -0.5
NEG = -1e9
LN2 = float(np.log(2.0))
S1 = NSEL + 1
RB = 256
NB = N // RB
JPAD = 128
GR = G * RB
KT = 512

_PERM = np.concatenate([np.arange(0, DH, 2), np.arange(1, DH, 2)])
_GPERM = np.array([h * 3 + j for j in range(3) for h in range(H)])


def _proj_kernel(x_ref, nw_ref, w_ref, bs_ref, cos_ref, sin_ref,
                 q_ref, k_ref, kpre_ref, v_ref, g_ref):
    xb = x_ref[...]
    ms = jnp.mean(xb * xb, axis=1, keepdims=True)
    xn = xb * lax.rsqrt(ms + 1e-6) * nw_ref[...]
    y = jnp.dot(xn, w_ref[...], preferred_element_type=jnp.float32)
    c = cos_ref[...]
    s = sin_ref[...]
    for h in range(H):
        a = y[:, h * DH:h * DH + DH // 2]
        b = y[:, h * DH + DH // 2:(h + 1) * DH]
        q_ref[h // G, h % G] = jnp.concatenate(
            [a * c - b * s, b * c + a * s], axis=1)
    for h in range(KH):
        base = H * DH + h * DH
        a = y[:, base:base + DH // 2]
        b = y[:, base + DH // 2:base + DH]
        k_ref[h] = jnp.concatenate([a * c - b * s, b * c + a * s], axis=1)
    kp0 = H * DH + KH * DH
    for h in range(KH):
        kpre_ref[h] = y[:, kp0 + h * DH:kp0 + (h + 1) * DH]
        v_ref[h] = y[:, kp0 + KH * DH + h * DH:kp0 + KH * DH + (h + 1) * DH]
    g0 = kp0 + 2 * KH * DH
    gy = jax.nn.sigmoid(y[:, g0:] + bs_ref[...])
    for j in range(3):
        for h in range(KH):
            g_ref[j, h] = gy[:, j * H + h * G:j * H + (h + 1) * G]


def _mlp_kernel(x_ref, pos_ref, w1_ref, b1_ref, w2_ref, b2_ref, o_ref):
    pos = jnp.concatenate(
        [jnp.broadcast_to(pos_ref[h:h + 1], (W, CBS * DH)) for h in range(KH)],
        axis=0)
    xp = x_ref[...] + pos
    hdn = jnp.maximum(
        jnp.dot(xp, w1_ref[...], preferred_element_type=jnp.float32)
        + b1_ref[...], 0.0)
    o_ref[...] = jnp.dot(hdn, w2_ref[...],
                         preferred_element_type=jnp.float32) + b2_ref[...]


def _mega_kernel(q_ref, k_ref, v_ref, ck_ref, cv_ref, gt_ref, wo_ref, o_ref):
    qb = pl.program_id(0)
    kh = pl.program_id(1)
    q2 = q_ref[0].reshape(GR, DH) * SCALE
    rows = qb * RB + lax.broadcasted_iota(jnp.int32, (RB, 1), 0)

    ck = ck_ref[0]
    cv = cv_ref[0]
    j = lax.broadcasted_iota(jnp.int32, (RB, JPAD), 1)
    cvalid = (j >= JPAD - NMEM) | ((j < W) & (rows >= (j + 1) * CBS - 1))
    csim = jnp.dot(q2, ck.T, preferred_element_type=jnp.float32)
    csim = jnp.where(cvalid[None], csim.reshape(G, RB, JPAD),
                     NEG).reshape(GR, JPAD)
    cmx = jnp.max(csim, axis=1, keepdims=True)
    cp = jnp.exp(csim - cmx)
    cattn = cp / jnp.sum(cp, axis=1, keepdims=True)
    cout = jnp.dot(cattn, cv,
                   preferred_element_type=jnp.float32).reshape(G, RB, DH)
    imp = jnp.sum(cattn.reshape(G, RB, JPAD)[:, :, :W], axis=0)
    lane_w = lax.broadcasted_iota(jnp.int32, (RB, W), 1)
    work = imp
    sels = []
    for _ in range(NSEL):
        mx = jnp.max(work, axis=1, keepdims=True)
        am = jnp.min(jnp.where(work == mx, lane_w, W), axis=1, keepdims=True)
        sels.append(am)
        work = jnp.where(lane_w == am, -jnp.inf, work)
    sels.append(rows // SBS)

    lane_t = lax.broadcasted_iota(jnp.int32, (1, KT), 1)
    nt = (qb + 2) // 2
    e_end = (qb + 1) * RB

    def flash_update(sim, vv, m, l, acc):
        mn = jnp.maximum(m, jnp.max(sim, axis=1, keepdims=True))
        p = jnp.exp(sim - mn)
        alpha = jnp.exp(m - mn)
        l = l * alpha + jnp.sum(p, axis=1, keepdims=True)
        acc = acc * alpha + jnp.dot(p, vv, preferred_element_type=jnp.float32)
        return mn, l, acc

    def fine_bias(kpos, limit):
        wl = kpos // SBS
        mult = jnp.zeros((RB, KT), jnp.int32)
        for s_i in range(S1):
            mult = mult + (sels[s_i] == wl).astype(jnp.int32)
        return jnp.where((mult > 0) & (kpos <= rows) & (kpos < limit),
                         jnp.where(mult == 2, LN2, 0.0), NEG)

    def fine_body(i, carry):
        m, l, acc = carry
        start_raw = e_end - KT * (nt - i)
        start = pl.multiple_of(jnp.maximum(start_raw, 0), RB)
        kk = k_ref[0, pl.ds(start, KT), :]
        vv = v_ref[0, pl.ds(start, KT), :]
        sim = jnp.dot(q2, kk.T, preferred_element_type=jnp.float32)
        kpos = start + lane_t
        bias = fine_bias(kpos, start_raw + KT)
        sim = (sim.reshape(G, RB, KT) + bias[None]).reshape(GR, KT)
        return flash_update(sim, vv, m, l, acc)

    init = (jnp.full((GR, 1), -1e30, jnp.float32),
            jnp.zeros((GR, 1), jnp.float32),
            jnp.zeros((GR, DH), jnp.float32))
    m, l, acc = lax.fori_loop(0, nt - 1, fine_body, init)

    start = pl.multiple_of(jnp.maximum(e_end - KT, 0), RB)
    kk = k_ref[0, pl.ds(start, KT), :]
    vv = v_ref[0, pl.ds(start, KT), :]
    tsim = jnp.dot(q2, kk.T, preferred_element_type=jnp.float32)
    kpos = start + lane_t
    bias = fine_bias(kpos, e_end)
    fsim = (tsim.reshape(G, RB, KT) + bias[None]).reshape(GR, KT)
    m, l, acc = flash_update(fsim, vv, m, l, acc)
    fout = (acc / l).reshape(G, RB, DH)

    dgap = rows - kpos
    wbias = jnp.where((dgap >= 0) & (dgap <= WIN), 0.0, NEG)
    wsim = (tsim.reshape(G, RB, KT) + wbias[None]).reshape(GR, KT)
    wmx = jnp.max(wsim, axis=1, keepdims=True)
    wp = jnp.exp(wsim - wmx)
    sout = (jnp.dot(wp, vv, preferred_element_type=jnp.float32)
            / jnp.sum(wp, axis=1, keepdims=True)).reshape(G, RB, DH)

    gt = gt_ref[:, 0]
    combs = []
    for g in range(G):
        combs.append(gt[0][:, g:g + 1] * cout[g]
                     + gt[1][:, g:g + 1] * fout[g]
                     + gt[2][:, g:g + 1] * sout[g])
    comb = jnp.concatenate(combs, axis=1)
    part = jnp.dot(comb, wo_ref[0], preferred_element_type=jnp.float32)

    @pl.when(kh == 0)
    def _():
        o_ref[...] = part

    @pl.when(kh != 0)
    def _():
        o_ref[...] += part


def kernel(hidden_states, norm_w, Wq, Wk, Wv, k_pos, v_pos, Wk1, bk1, Wk2,
           bk2, Wv1, bv1, Wv2, bv2, mem_kv, Ws, bs, Wo):
    P = _PERM
    x = hidden_states.reshape(N, D)
    WqP = Wq.reshape(D, H, DH)[:, :, P].reshape(D, H * DH)
    WkP = Wk.reshape(D, KH, DH)[:, :, P].reshape(D, KH * DH)
    Wcat = jnp.concatenate([WqP, WkP, Wk, Wv, Ws[:, _GPERM]], axis=1)
    CW = H * DH + 3 * KH * DH + 3 * H

    posf = jnp.arange(N, dtype=jnp.float32)
    inv = 1.0 / (10000.0 ** (jnp.arange(0, DH, 2, dtype=jnp.float32) / DH))
    ang = posf[:, None] * inv[None, :]
    cosT = jnp.cos(ang)
    sinT = jnp.sin(ang)

    q4, krot, kpre, vkh, gates = pl.pallas_call(
        _proj_kernel,
        grid=(NB,),
        in_specs=[
            pl.BlockSpec((RB, D), lambda i: (i, 0)),
            pl.BlockSpec((1, D), lambda i: (0, 0)),
            pl.BlockSpec((D, CW), lambda i: (0, 0)),
            pl.BlockSpec((1, 3 * H), lambda i: (0, 0)),
            pl.BlockSpec((RB, DH // 2), lambda i: (i, 0)),
            pl.BlockSpec((RB, DH // 2), lambda i: (i, 0)),
        ],
        out_specs=[
            pl.BlockSpec((KH, G, RB, DH), lambda i: (0, 0, i, 0)),
            pl.BlockSpec((KH, RB, DH), lambda i: (0, i, 0)),
            pl.BlockSpec((KH, RB, DH), lambda i: (0, i, 0)),
            pl.BlockSpec((KH, RB, DH), lambda i: (0, i, 0)),
            pl.BlockSpec((3, KH, RB, G), lambda i: (0, 0, i, 0)),
        ],
        out_shape=[
            jax.ShapeDtypeStruct((KH, G, N, DH), jnp.float32),
            jax.ShapeDtypeStruct((KH, N, DH), jnp.float32),
            jax.ShapeDtypeStruct((KH, N, DH), jnp.float32),
            jax.ShapeDtypeStruct((KH, N, DH), jnp.float32),
            jax.ShapeDtypeStruct((3, KH, N, G), jnp.float32),
        ],
    )(x, norm_w.reshape(1, D), Wcat, bs[_GPERM].reshape(1, 3 * H), cosT, sinT)

    def mlp_call(xflat, pos2, W1, b1, W2, b2):
        return pl.pallas_call(
            _mlp_kernel,
            grid=(1,),
            in_specs=[
                pl.BlockSpec((KH * W, CBS * DH), lambda i: (0, 0)),
                pl.BlockSpec((KH, CBS * DH), lambda i: (0, 0)),
                pl.BlockSpec((CBS * DH, CBS * DH), lambda i: (0, 0)),
                pl.BlockSpec((1, CBS * DH), lambda i: (0, 0)),
                pl.BlockSpec((CBS * DH, DH), lambda i: (0, 0)),
                pl.BlockSpec((1, DH), lambda i: (0, 0)),
            ],
            out_specs=pl.BlockSpec((KH * W, DH), lambda i: (0, 0)),
            out_shape=jax.ShapeDtypeStruct((KH * W, DH), jnp.float32),
        )(xflat, pos2, W1, b1.reshape(1, CBS * DH), W2, b2.reshape(1, DH))

    kflat = kpre.reshape(KH * W, CBS * DH)
    vflat = vkh.reshape(KH * W, CBS * DH)
    ck = mlp_call(kflat, k_pos.reshape(KH, CBS * DH), Wk1, bk1, Wk2,
                  bk2).reshape(KH, W, DH)
    cv = mlp_call(vflat, v_pos.reshape(KH, CBS * DH), Wv1, bv1, Wv2,
                  bv2).reshape(KH, W, DH)

    zpad = jnp.zeros((KH, JPAD - NMEM - W, DH), jnp.float32)
    ckf = jnp.concatenate([ck, zpad, mem_kv[0]], axis=1)[:, :, P]
    cvf = jnp.concatenate([cv, zpad, mem_kv[1]], axis=1)

    out = pl.pallas_call(
        _mega_kernel,
        grid=(NB, KH),
        in_specs=[
            pl.BlockSpec((1, G, RB, DH), lambda i, h: (h, 0, i, 0)),
            pl.BlockSpec((1, N, DH), lambda i, h: (h, 0, 0)),
            pl.BlockSpec((1, N, DH), lambda i, h: (h, 0, 0)),
            pl.BlockSpec((1, JPAD, DH), lambda i, h: (h, 0, 0)),
            pl.BlockSpec((1, JPAD, DH), lambda i, h: (h, 0, 0)),
            pl.BlockSpec((3, 1, RB, G), lambda i, h: (0, h, i, 0)),
            pl.BlockSpec((1, G * DH, D), lambda i, h: (h, 0, 0)),
        ],
        out_specs=pl.BlockSpec((RB, D), lambda i, h: (i, 0)),
        out_shape=jax.ShapeDtypeStruct((N, D), jnp.float32),
    )(q4, krot, vkh, ckf, cvf, gates, Wo.reshape(KH, G * DH, D))
    return out.reshape(B, N, D)
